# baseline (speedup 1.0000x reference)
import functools

import jax
import jax.numpy as jnp
from jax import lax
from jax.experimental import pallas as pl
from jax.experimental.pallas import tpu as pltpu
from jax.experimental.pallas import tpu_sc as plsc

N = 10000
D = 128
K = 32
L = 16

NC = 2
NS = 16
NW = NC * NS

R = 320
G = 4
GK = G * K
NCH = R // G

_mesh = plsc.VectorSubcoreMesh(core_axis_name="c", subcore_axis_name="s")


@functools.partial(
    pl.kernel,
    out_type=jax.ShapeDtypeStruct((N, D), jnp.float32),
    mesh=_mesh,
    scratch_types=[
        pltpu.VMEM((R * K,), jnp.int32),
        pltpu.VMEM((GK, D), jnp.float32),
        pltpu.VMEM((GK, D), jnp.float32),
        pltpu.VMEM((R, D), jnp.float32),
        pltpu.SemaphoreType.DMA,
        pltpu.SemaphoreType.DMA,
    ],
)
def _maxpool_sc(feats_hbm, idx_hbm, out_hbm, idx_v, nb0, nb1, out_v,
                sem0, sem1):
    wid = lax.axis_index("s") * NC + lax.axis_index("c")
    base = jnp.minimum(wid * R, N - R)

    pltpu.sync_copy(idx_hbm.at[pl.ds(base * K, R * K)], idx_v)

    def fire(ch, nb, sem):
        pltpu.make_async_copy(
            feats_hbm.at[idx_v.at[pl.ds(ch * GK, GK)]], nb, sem).start()

    def drain(ch, nb, sem):
        pltpu.make_async_copy(
            feats_hbm.at[idx_v.at[pl.ds(ch * GK, GK)]], nb, sem).wait()

    def reduce_chunk(nb, ch):
        for g in range(G):
            row = ch * G + g
            for c in range(D // L):
                sl = pl.ds(c * L, L)
                halves = []
                for h in range(2):
                    vals = [nb[g * K + h * 16 + k, sl] for k in range(16)]
                    while len(vals) > 1:
                        vals = [jnp.maximum(vals[i], vals[i + 1])
                                for i in range(0, len(vals), 2)]
                    halves.append(vals[0])
                out_v[row, sl] = jnp.maximum(halves[0], halves[1])

    bufs = ((nb0, sem0), (nb1, sem1))
    NB = len(bufs)

    for b in range(NB - 1):
        fire(b, *bufs[b])

    @pl.loop(0, NCH, step=NB)
    def _(ch):
        for b in range(NB):
            nxt = ch + b + NB - 1

            @pl.when(nxt < NCH)
            def _(nxt=nxt, b=b):
                fire(nxt, *bufs[(b + NB - 1) % NB])

            drain(ch + b, *bufs[b])
            reduce_chunk(bufs[b][0], ch + b)

    pltpu.sync_copy(out_v, out_hbm.at[pl.ds(base, R)])


def kernel(s_feats, neighbor_indices):
    idx_flat = neighbor_indices.astype(jnp.int32).reshape(-1)
    return _maxpool_sc(s_feats, idx_flat)


# speedup vs baseline: 1.3601x; 1.3601x over previous
import functools

import jax
import jax.numpy as jnp
from jax import lax
from jax.experimental import pallas as pl
from jax.experimental.pallas import tpu as pltpu
from jax.experimental.pallas import tpu_sc as plsc

N = 10000
D = 128
K = 32
L = 16

NC = 2
NS = 16
NW = NC * NS

R = 320
G = 4
GK = G * K
NCH = R // G

_mesh = plsc.VectorSubcoreMesh(core_axis_name="c", subcore_axis_name="s")


@functools.partial(
    pl.kernel,
    out_type=jax.ShapeDtypeStruct((N, D), jnp.float32),
    mesh=_mesh,
    scratch_types=[
        pltpu.VMEM((R * K,), jnp.int32),
        pltpu.VMEM((GK, D), jnp.float32),
        pltpu.VMEM((GK, D), jnp.float32),
        pltpu.VMEM((R, D), jnp.float32),
        pltpu.SemaphoreType.DMA,
        pltpu.SemaphoreType.DMA,
    ],
)
def _maxpool_sc(feats_hbm, idx_hbm, out_hbm, idx_v, nb0, nb1, out_v,
                sem0, sem1):
    wid = lax.axis_index("s") * NC + lax.axis_index("c")
    base = jnp.minimum(wid * R, N - R)

    pltpu.sync_copy(idx_hbm.at[pl.ds(base * K, R * K)], idx_v)

    def fire(ch, nb, sem):
        pltpu.make_async_copy(
            feats_hbm.at[idx_v.at[pl.ds(ch * GK, GK)]], nb, sem).start()

    def drain(ch, nb, sem):
        pltpu.make_async_copy(
            feats_hbm.at[idx_v.at[pl.ds(ch * GK, GK)]], nb, sem).wait()

    def reduce_chunk(nb, ch):
        NCHAIN = 2
        for g in range(G):
            row = ch * G + g
            for half in range(D // L // NCHAIN):
                cs = range(half * NCHAIN, (half + 1) * NCHAIN)
                acc = {c: nb[g * K, pl.ds(c * L, L)] for c in cs}
                for k in range(1, K):
                    for c in cs:
                        acc[c] = jnp.maximum(acc[c], nb[g * K + k, pl.ds(c * L, L)])
                for c in cs:
                    out_v[row, pl.ds(c * L, L)] = acc[c]

    bufs = ((nb0, sem0), (nb1, sem1))
    NB = len(bufs)

    for b in range(NB - 1):
        fire(b, *bufs[b])

    @pl.loop(0, NCH, step=NB)
    def _(ch):
        plsc.subcore_barrier()
        for b in range(NB):
            nxt = ch + b + NB - 1

            @pl.when(nxt < NCH)
            def _(nxt=nxt, b=b):
                fire(nxt, *bufs[(b + NB - 1) % NB])

            drain(ch + b, *bufs[b])
            reduce_chunk(bufs[b][0], ch + b)

    pltpu.sync_copy(out_v, out_hbm.at[pl.ds(base, R)])


def kernel(s_feats, neighbor_indices):
    idx_flat = neighbor_indices.astype(jnp.int32).reshape(-1)
    return _maxpool_sc(s_feats, idx_flat)


# speedup vs baseline: 1.3780x; 1.0132x over previous
import functools

import jax
import jax.numpy as jnp
from jax import lax
from jax.experimental import pallas as pl
from jax.experimental.pallas import tpu as pltpu
from jax.experimental.pallas import tpu_sc as plsc

N = 10000
D = 128
K = 32
L = 16

NC = 2
NS = 16
NW = NC * NS

R = 320
G = 4
GK = G * K
NCH = R // G

_mesh = plsc.VectorSubcoreMesh(core_axis_name="c", subcore_axis_name="s")


@functools.partial(
    pl.kernel,
    out_type=jax.ShapeDtypeStruct((N, D), jnp.float32),
    mesh=_mesh,
    scratch_types=[
        pltpu.VMEM((R * K,), jnp.int32),
        pltpu.VMEM((GK, D), jnp.float32),
        pltpu.VMEM((GK, D), jnp.float32),
        pltpu.VMEM((R, D), jnp.float32),
        pltpu.SemaphoreType.DMA,
        pltpu.SemaphoreType.DMA,
    ],
)
def _maxpool_sc(feats_hbm, idx_hbm, out_hbm, idx_v, nb0, nb1, out_v,
                sem0, sem1):
    wid = lax.axis_index("s") * NC + lax.axis_index("c")
    base = jnp.minimum(wid * R, N - R)

    pltpu.sync_copy(idx_hbm.at[pl.ds(base * K, R * K)], idx_v)

    def fire(ch, nb, sem):
        pltpu.make_async_copy(
            feats_hbm.at[idx_v.at[pl.ds(ch * GK, GK)]], nb, sem).start()

    def drain(ch, nb, sem):
        pltpu.make_async_copy(
            feats_hbm.at[idx_v.at[pl.ds(ch * GK, GK)]], nb, sem).wait()

    def reduce_chunk(nb, ch):
        NCHAIN = 2
        for g in range(G):
            row = ch * G + g
            for half in range(D // L // NCHAIN):
                cs = range(half * NCHAIN, (half + 1) * NCHAIN)
                acc = {c: nb[g * K, pl.ds(c * L, L)] for c in cs}
                for k in range(1, K):
                    for c in cs:
                        acc[c] = jnp.maximum(acc[c], nb[g * K + k, pl.ds(c * L, L)])
                for c in cs:
                    out_v[row, pl.ds(c * L, L)] = acc[c]

    bufs = ((nb0, sem0), (nb1, sem1))
    NB = len(bufs)

    for b in range(NB - 1):
        fire(b, *bufs[b])

    @pl.loop(0, NCH, step=NB)
    def _(ch):
        for b in range(NB):
            plsc.subcore_barrier()
            nxt = ch + b + NB - 1

            @pl.when(nxt < NCH)
            def _(nxt=nxt, b=b):
                fire(nxt, *bufs[(b + NB - 1) % NB])

            drain(ch + b, *bufs[b])
            reduce_chunk(bufs[b][0], ch + b)

    pltpu.sync_copy(out_v, out_hbm.at[pl.ds(base, R)])


def kernel(s_feats, neighbor_indices):
    idx_flat = neighbor_indices.astype(jnp.int32).reshape(-1)
    return _maxpool_sc(s_feats, idx_flat)


# speedup vs baseline: 1.5392x; 1.1170x over previous
import functools

import jax
import jax.numpy as jnp
from jax import lax
from jax.experimental import pallas as pl
from jax.experimental.pallas import tpu as pltpu
from jax.experimental.pallas import tpu_sc as plsc

N = 10000
D = 128
K = 32
L = 16

NC = 2
NS = 16
NW = NC * NS

R = 320
G = 4
GK = G * K
NCH = R // G

_mesh = plsc.VectorSubcoreMesh(core_axis_name="c", subcore_axis_name="s")


@functools.partial(
    pl.kernel,
    out_type=jax.ShapeDtypeStruct((N, D), jnp.float32),
    mesh=_mesh,
    scratch_types=[
        pltpu.VMEM((R * K,), jnp.int32),
        pltpu.VMEM((GK, D), jnp.float32),
        pltpu.VMEM((GK, D), jnp.float32),
        pltpu.VMEM((R, D), jnp.float32),
        pltpu.SemaphoreType.DMA,
        pltpu.SemaphoreType.DMA,
    ],
)
def _maxpool_sc(feats_hbm, idx_hbm, out_hbm, idx_v, nb0, nb1, out_v,
                sem0, sem1):
    wid = lax.axis_index("s") * NC + lax.axis_index("c")
    base = jnp.minimum(wid * R, N - R)

    pltpu.sync_copy(idx_hbm.at[pl.ds(base * K, R * K)], idx_v)

    def fire(ch, nb, sem):
        pltpu.make_async_copy(
            feats_hbm.at[idx_v.at[pl.ds(ch * GK, GK)]], nb, sem).start()

    def drain(ch, nb, sem):
        pltpu.make_async_copy(
            feats_hbm.at[idx_v.at[pl.ds(ch * GK, GK)]], nb, sem).wait()

    def reduce_chunk(nb, ch):
        NCHAIN = 2
        for g in range(G):
            plsc.subcore_barrier()
            row = ch * G + g
            for half in range(D // L // NCHAIN):
                cs = range(half * NCHAIN, (half + 1) * NCHAIN)
                acc = {c: nb[g * K, pl.ds(c * L, L)] for c in cs}
                for k in range(1, K):
                    for c in cs:
                        acc[c] = jnp.maximum(acc[c], nb[g * K + k, pl.ds(c * L, L)])
                for c in cs:
                    out_v[row, pl.ds(c * L, L)] = acc[c]

    bufs = ((nb0, sem0), (nb1, sem1))
    NB = len(bufs)

    for b in range(NB - 1):
        fire(b, *bufs[b])

    @pl.loop(0, NCH, step=NB)
    def _(ch):
        for b in range(NB):
            plsc.subcore_barrier()
            nxt = ch + b + NB - 1

            @pl.when(nxt < NCH)
            def _(nxt=nxt, b=b):
                fire(nxt, *bufs[(b + NB - 1) % NB])

            drain(ch + b, *bufs[b])
            reduce_chunk(bufs[b][0], ch + b)

    pltpu.sync_copy(out_v, out_hbm.at[pl.ds(base, R)])


def kernel(s_feats, neighbor_indices):
    idx_flat = neighbor_indices.astype(jnp.int32).reshape(-1)
    return _maxpool_sc(s_feats, idx_flat)
